# baseline (speedup 1.0000x reference)
import functools
import jax
import jax.numpy as jnp
from jax import lax
from jax.experimental import pallas as pl
from jax.experimental.pallas import tpu as pltpu, tpu_sc as plsc

C = 128
HC = C // 2
N_NODE = 10000
N_EDGE = 320000
NC, NS, L = 2, 16, 16
NW = NC * NS
CH = 80
EPW = N_EDGE // NW
NCK_W = EPW // CH
EPS = N_EDGE // NS
NCK_S = EPS // CH
RP = 624
ZR = 208
INV_SQRT_C = 1.0 / (C ** 0.5)

_mesh = plsc.VectorSubcoreMesh(core_axis_name="c", subcore_axis_name="s")
_sc_params = pltpu.CompilerParams(use_tc_tiling_on_sc=False, has_side_effects=True)
_sc_params_nl = pltpu.CompilerParams(use_tc_tiling_on_sc=False,
                                     needs_layout_passes=False,
                                     has_side_effects=True)


def _zero_vmem_rows(ref, nrows, ncols):
    @pl.loop(0, nrows)
    def _(i):
        for j in range(ncols // L):
            ref[i, pl.ds(j * L, L)] = jnp.zeros((L,), jnp.float32)


def _zero_shared_rows(zbuf, sh_ref, sub):
    for t in range(RP // ZR):
        pltpu.sync_copy(zbuf, sh_ref.at[pl.ds(sub * RP + t * ZR, ZR)])

    @pl.when(sub == NS - 1)
    def _():
        pltpu.sync_copy(zbuf.at[pl.ds(0, L)], sh_ref.at[pl.ds(RP * NS, L)])


def _copy_out_rows(sh_ref, out_ref, sub):
    pltpu.sync_copy(sh_ref.at[pl.ds(sub * RP, RP)],
                    out_ref.at[pl.ds(sub * RP, RP)])

    @pl.when(sub == NS - 1)
    def _():
        pltpu.sync_copy(sh_ref.at[pl.ds(RP * NS, L)],
                        out_ref.at[pl.ds(RP * NS, L)])


@functools.partial(
    pl.kernel,
    out_type=[
        jax.ShapeDtypeStruct((NC, N_NODE, HC), jnp.float32),
        jax.ShapeDtypeStruct((N_NODE, L), jnp.float32),
    ],
    mesh=_mesh,
    compiler_params=_sc_params,
    scratch_types=[
        pltpu.VMEM((NCK_S, CH), jnp.int32),
        pltpu.VMEM((NCK_S, CH), jnp.int32),
        pltpu.VMEM((CH, HC), jnp.float32),
        pltpu.VMEM((CH, L), jnp.float32),
        pltpu.VMEM((ZR, HC), jnp.float32),
        pltpu.VMEM((ZR, L), jnp.float32),
        pltpu.VMEM_SHARED((N_NODE, HC), jnp.float32),
        pltpu.VMEM_SHARED((N_NODE, L), jnp.float32),
        pltpu.SemaphoreType.DMA,
    ],
)
def _seg_sum_kernel(src3d, mid3d, xsrc_sp, s_out, deg_out,
                    idx1_v, idx2_v, rows_v, ones_v, zbuf, degz,
                    s_sh, deg_sh, sem):
    core = lax.axis_index("c")
    sub = lax.axis_index("s")

    _zero_vmem_rows(zbuf, ZR, HC)
    _zero_shared_rows(zbuf, s_sh, sub)

    @pl.when(core == 0)
    def _():
        _zero_vmem_rows(degz, ZR, L)
        for t in range(RP // ZR):
            pltpu.sync_copy(degz, deg_sh.at[pl.ds(sub * RP + t * ZR, ZR)])

        @pl.when(sub == NS - 1)
        def _():
            pltpu.sync_copy(degz.at[pl.ds(0, L)], deg_sh.at[pl.ds(RP * NS, L)])

        @pl.loop(0, CH)
        def _(i):
            ones_v[i, :] = jnp.ones((L,), jnp.float32)

    plsc.subcore_barrier()

    pltpu.sync_copy(src3d.at[sub], idx1_v)
    pltpu.sync_copy(mid3d.at[sub], idx2_v)

    @pl.loop(0, NCK_S)
    def _(c):
        pltpu.async_copy(xsrc_sp.at[core].at[idx1_v.at[c]], rows_v, sem).wait()
        pltpu.sync_copy(rows_v, s_sh.at[idx2_v.at[c]], add=True)

        @pl.when(core == 0)
        def _():
            pltpu.sync_copy(ones_v, deg_sh.at[idx2_v.at[c]], add=True)

    plsc.subcore_barrier()

    _copy_out_rows(s_sh, s_out.at[core], sub)

    @pl.when(core == 0)
    def _():
        _copy_out_rows(deg_sh, deg_out, sub)


_RB = 1000


def _dense_body(xmid, xdst, s0, s1, deg16, W1, b12, Wq, bq, Wk, bk, Wv, bv,
                q_o, k_o, v_o):
    deg = deg16[:, 0:1]
    s = jnp.concatenate([s0[...], s1[...]], axis=1)
    agg = deg * (jnp.dot(xmid[...], W1[...], preferred_element_type=jnp.float32)
                 + b12[...]) + s
    q_o[...] = jnp.dot(xdst[...], Wq[...], preferred_element_type=jnp.float32) + bq[...]
    k_o[...] = jnp.dot(agg, Wk[...], preferred_element_type=jnp.float32) + bk[...]
    v_o[...] = jnp.dot(agg, Wv[...], preferred_element_type=jnp.float32) + bv[...]


def _dense_qkv(xmid, xdst, s0, s1, deg16, W1, b12, Wq, bq, Wk, bk, Wv, bv):
    row_spec = pl.BlockSpec((_RB, C), lambda i: (i, 0))
    half_spec = pl.BlockSpec((_RB, HC), lambda i: (i, 0))
    deg_spec = pl.BlockSpec((_RB, L), lambda i: (i, 0))
    w_spec = pl.BlockSpec((C, C), lambda i: (0, 0))
    b_spec = pl.BlockSpec((C,), lambda i: (0,))
    return pl.pallas_call(
        _dense_body,
        grid=(N_NODE // _RB,),
        in_specs=[row_spec, row_spec, half_spec, half_spec, deg_spec,
                  w_spec, b_spec, w_spec, b_spec, w_spec, b_spec,
                  w_spec, b_spec],
        out_specs=[row_spec, row_spec, row_spec],
        out_shape=[jax.ShapeDtypeStruct((N_NODE, C), jnp.float32)] * 3,
    )(xmid, xdst, s0, s1, deg16, W1, b12, Wq, bq, Wk, bk, Wv, bv)


def _xw_body(x, W, o):
    o[...] = jnp.dot(x[...], W[...], preferred_element_type=jnp.float32)


def _xw(x, W):
    row_spec = pl.BlockSpec((_RB, C), lambda i: (i, 0))
    w_spec = pl.BlockSpec((C, C), lambda i: (0, 0))
    return pl.pallas_call(
        _xw_body,
        grid=(N_NODE // _RB,),
        in_specs=[row_spec, w_spec],
        out_specs=row_spec,
        out_shape=jax.ShapeDtypeStruct((N_NODE, C), jnp.float32),
    )(x, W)


@functools.partial(
    pl.kernel,
    out_type=[
        jax.ShapeDtypeStruct((NW, NCK_W, CH), jnp.float32),
        jax.ShapeDtypeStruct((NW, 1, L), jnp.float32),
        jax.ShapeDtypeStruct((NW, 1, L), jnp.float32),
    ],
    mesh=_mesh,
    compiler_params=_sc_params_nl,
    scratch_types=[
        pltpu.VMEM((NCK_W, CH), jnp.int32),
        pltpu.VMEM((NCK_W, CH), jnp.int32),
        pltpu.VMEM((2, CH, C), jnp.float32),
        pltpu.VMEM((2, CH, C), jnp.float32),
        pltpu.VMEM((NCK_W, CH), jnp.float32),
        pltpu.VMEM((1, L), jnp.float32),
        pltpu.VMEM((1, L), jnp.float32),
        pltpu.SemaphoreType.DMA,
        pltpu.SemaphoreType.DMA,
        pltpu.SemaphoreType.DMA,
        pltpu.SemaphoreType.DMA,
    ],
)
def _logits_kernel(mid3d, dst3d, qtab, ktab, lg_out, mx_out, se_out,
                   idxm_v, idxd_v, q_bufs, k_bufs, lg_v, mstat, sstat,
                   qs0, qs1, ks0, ks1):
    core = lax.axis_index("c")
    sub = lax.axis_index("s")
    wid = sub * NC + core
    lane = jnp.arange(L, dtype=jnp.int32)
    qsems = (qs0, qs1)
    ksems = (ks0, ks1)

    pltpu.sync_copy(mid3d.at[wid], idxm_v)
    pltpu.sync_copy(dst3d.at[wid], idxd_v)

    def _start(c, b):
        pltpu.async_copy(qtab.at[idxd_v.at[c]], q_bufs.at[b], qsems[b])
        pltpu.async_copy(ktab.at[idxm_v.at[c]], k_bufs.at[b], ksems[b])

    def _wait(b):
        pltpu.make_async_copy(qtab.at[idxd_v.at[0]], q_bufs.at[b],
                              qsems[b]).wait()
        pltpu.make_async_copy(ktab.at[idxm_v.at[0]], k_bufs.at[b],
                              ksems[b]).wait()

    def _compute(c, b):
        q_rows = q_bufs.at[b]
        k_rows = k_bufs.at[b]
        for g in range(CH // L):
            rowsel = jnp.full((L,), g * L, jnp.int32) + lane
            dots = jnp.zeros((L,), jnp.float32)
            for j in range(C):
                colsel = jnp.full((L,), j, jnp.int32)
                qc = plsc.load_gather(q_rows, [rowsel, colsel])
                kc = plsc.load_gather(k_rows, [rowsel, colsel])
                dots = dots + qc * kc
            lg_v[c, pl.ds(g * L, L)] = dots * INV_SQRT_C

    _start(0, 0)
    _start(1, 1)

    @pl.loop(0, NCK_W - 1, step=2)
    def _(c):
        for b in range(2):
            _wait(b)
            _compute(c + b, b)

            @pl.when(c + 2 + b < NCK_W)
            def _():
                _start(c + 2 + b, b)

    _wait((NCK_W - 1) % 2)
    _compute(NCK_W - 1, (NCK_W - 1) % 2)

    def mx_body(c, m):
        for g in range(CH // L):
            m = jnp.maximum(m, lg_v[c, pl.ds(g * L, L)])
        return m

    m_vec = lax.fori_loop(0, NCK_W, mx_body,
                          jnp.full((L,), -jnp.inf, jnp.float32))
    m_loc = jnp.max(m_vec)

    def se_body(c, s):
        for g in range(CH // L):
            s = s + jnp.exp(lg_v[c, pl.ds(g * L, L)] - m_loc)
        return s

    s_vec = lax.fori_loop(0, NCK_W, se_body, jnp.zeros((L,), jnp.float32))
    s_loc = jnp.sum(s_vec)

    mstat[0, :] = jnp.zeros((L,), jnp.float32) + m_loc
    sstat[0, :] = jnp.zeros((L,), jnp.float32) + s_loc
    pltpu.sync_copy(lg_v, lg_out.at[wid])
    pltpu.sync_copy(mstat, mx_out.at[wid])
    pltpu.sync_copy(sstat, se_out.at[wid])


@functools.partial(
    pl.kernel,
    out_type=jax.ShapeDtypeStruct((NC, N_NODE, HC), jnp.float32),
    mesh=_mesh,
    compiler_params=_sc_params,
    scratch_types=[
        pltpu.VMEM((NCK_S, CH), jnp.int32),
        pltpu.VMEM((NCK_S, CH), jnp.int32),
        pltpu.VMEM((NCK_S, CH), jnp.float32),
        pltpu.VMEM((CH, HC), jnp.float32),
        pltpu.VMEM((NW, 1, L), jnp.float32),
        pltpu.VMEM((NW, 1, L), jnp.float32),
        pltpu.VMEM((ZR, HC), jnp.float32),
        pltpu.VMEM_SHARED((N_NODE, HC), jnp.float32),
        pltpu.SemaphoreType.DMA,
    ],
)
def _attn_out_kernel(mid3d, dst3d, lg3d, vtab_sp, mx_in, se_in, out,
                     idxm_v, idxd_v, lg_v, v_rows, mx_v, se_v, zbuf,
                     o_sh, sem):
    core = lax.axis_index("c")
    sub = lax.axis_index("s")

    _zero_vmem_rows(zbuf, ZR, HC)
    _zero_shared_rows(zbuf, o_sh, sub)
    plsc.subcore_barrier()

    pltpu.sync_copy(mid3d.at[sub], idxm_v)
    pltpu.sync_copy(dst3d.at[sub], idxd_v)
    pltpu.sync_copy(lg3d.at[sub], lg_v)
    pltpu.sync_copy(mx_in, mx_v)
    pltpu.sync_copy(se_in, se_v)

    m_glob = mx_v[0, 0, :]
    for w in range(1, NW):
        m_glob = jnp.maximum(m_glob, mx_v[w, 0, :])
    z_vec = jnp.zeros((L,), jnp.float32)
    for w in range(NW):
        z_vec = z_vec + jnp.exp(mx_v[w, 0, :] - m_glob) * se_v[w, 0, :]
    inv_z = 1.0 / z_vec

    @pl.loop(0, NCK_S)
    def _(c):
        pltpu.async_copy(vtab_sp.at[core].at[idxm_v.at[c]], v_rows, sem).wait()
        for g in range(CH // L):
            a_g = jnp.exp(lg_v[c, pl.ds(g * L, L)] - m_glob) * inv_z
            for e in range(L):
                ee = g * L + e
                a_e = a_g[e]
                for j in range(HC // L):
                    v_rows[ee, pl.ds(j * L, L)] = v_rows[ee, pl.ds(j * L, L)] * a_e
        pltpu.sync_copy(v_rows, o_sh.at[idxd_v.at[c]], add=True)

    plsc.subcore_barrier()
    _copy_out_rows(o_sh, out.at[core], sub)


def _concat_body(a, b, o):
    o[...] = jnp.concatenate([a[...], b[...]], axis=1)


def _concat_halves(a, b):
    half_spec = pl.BlockSpec((_RB, HC), lambda i: (i, 0))
    return pl.pallas_call(
        _concat_body,
        grid=(N_NODE // _RB,),
        in_specs=[half_spec, half_spec],
        out_specs=pl.BlockSpec((_RB, C), lambda i: (i, 0)),
        out_shape=jax.ShapeDtypeStruct((N_NODE, C), jnp.float32),
    )(a, b)


def kernel(x_src, x_mid, x_dst, edge_index_1, edge_index_2,
           W1, b1, W2, b2, Wq, bq, Wk, bk, Wv, bv):
    src_s = edge_index_1[0].astype(jnp.int32).reshape(NS, NCK_S, CH)
    mid_s = edge_index_1[1].astype(jnp.int32).reshape(NS, NCK_S, CH)
    mid_w2 = edge_index_2[0].astype(jnp.int32).reshape(NW, NCK_W, CH)
    dst_w2 = edge_index_2[1].astype(jnp.int32).reshape(NW, NCK_W, CH)
    mid_s2 = edge_index_2[0].astype(jnp.int32).reshape(NS, NCK_S, CH)
    dst_s2 = edge_index_2[1].astype(jnp.int32).reshape(NS, NCK_S, CH)

    t2 = _xw(x_src, W2)
    t2_sp = jnp.stack([t2[:, :HC], t2[:, HC:]])

    s_halves, deg16 = _seg_sum_kernel(src_s, mid_s, t2_sp)

    q_tab, k_tab, v_tab = _dense_qkv(
        x_mid, x_dst, s_halves[0], s_halves[1], deg16,
        W1, b1 + b2, Wq, bq, Wk, bk, Wv, bv)

    lg3d, mx, se = _logits_kernel(mid_w2, dst_w2, q_tab, k_tab)

    vtab_sp = jnp.stack([v_tab[:, :HC], v_tab[:, HC:]])
    lg_s = lg3d.reshape(NS, NCK_S, CH)

    out_halves = _attn_out_kernel(mid_s2, dst_s2, lg_s, vtab_sp, mx, se)
    return _concat_halves(out_halves[0], out_halves[1])


# speedup vs baseline: 1.6169x; 1.6169x over previous
import functools
import jax
import jax.numpy as jnp
from jax import lax
from jax.experimental import pallas as pl
from jax.experimental.pallas import tpu as pltpu, tpu_sc as plsc

C = 128
HC = C // 2
N_NODE = 10000
N_EDGE = 320000
NC, NS, L = 2, 16, 16
NW = NC * NS
CH = 80
EPW = N_EDGE // NW
NCK_W = EPW // CH
EPS = N_EDGE // NS
NCK_S = EPS // CH
RP = 624
ZR = 208
INV_SQRT_C = 1.0 / (C ** 0.5)

_mesh = plsc.VectorSubcoreMesh(core_axis_name="c", subcore_axis_name="s")
_sc_params = pltpu.CompilerParams(use_tc_tiling_on_sc=False, has_side_effects=True)
_sc_params_nl = pltpu.CompilerParams(use_tc_tiling_on_sc=False,
                                     needs_layout_passes=False,
                                     has_side_effects=True)


def _zero_vmem_rows(ref, nrows, ncols):
    @pl.loop(0, nrows)
    def _(i):
        for j in range(ncols // L):
            ref[i, pl.ds(j * L, L)] = jnp.zeros((L,), jnp.float32)


def _zero_shared_rows(zbuf, sh_ref, sub):
    for t in range(RP // ZR):
        pltpu.sync_copy(zbuf, sh_ref.at[pl.ds(sub * RP + t * ZR, ZR)])

    @pl.when(sub == NS - 1)
    def _():
        pltpu.sync_copy(zbuf.at[pl.ds(0, L)], sh_ref.at[pl.ds(RP * NS, L)])


def _copy_out_rows(sh_ref, out_ref, sub):
    pltpu.sync_copy(sh_ref.at[pl.ds(sub * RP, RP)],
                    out_ref.at[pl.ds(sub * RP, RP)])

    @pl.when(sub == NS - 1)
    def _():
        pltpu.sync_copy(sh_ref.at[pl.ds(RP * NS, L)],
                        out_ref.at[pl.ds(RP * NS, L)])


@functools.partial(
    pl.kernel,
    out_type=[
        jax.ShapeDtypeStruct((NC, N_NODE, HC), jnp.float32),
        jax.ShapeDtypeStruct((N_NODE, L), jnp.float32),
    ],
    mesh=_mesh,
    compiler_params=_sc_params,
    scratch_types=[
        pltpu.VMEM((NCK_S, CH), jnp.int32),
        pltpu.VMEM((NCK_S, CH), jnp.int32),
        pltpu.VMEM((CH, HC), jnp.float32),
        pltpu.VMEM((CH, L), jnp.float32),
        pltpu.VMEM((ZR, HC), jnp.float32),
        pltpu.VMEM((ZR, L), jnp.float32),
        pltpu.VMEM_SHARED((N_NODE, HC), jnp.float32),
        pltpu.VMEM_SHARED((N_NODE, L), jnp.float32),
        pltpu.SemaphoreType.DMA,
    ],
)
def _seg_sum_kernel(src3d, mid3d, xsrc_sp, s_out, deg_out,
                    idx1_v, idx2_v, rows_v, ones_v, zbuf, degz,
                    s_sh, deg_sh, sem):
    core = lax.axis_index("c")
    sub = lax.axis_index("s")

    _zero_vmem_rows(zbuf, ZR, HC)
    _zero_shared_rows(zbuf, s_sh, sub)

    @pl.when(core == 0)
    def _():
        _zero_vmem_rows(degz, ZR, L)
        for t in range(RP // ZR):
            pltpu.sync_copy(degz, deg_sh.at[pl.ds(sub * RP + t * ZR, ZR)])

        @pl.when(sub == NS - 1)
        def _():
            pltpu.sync_copy(degz.at[pl.ds(0, L)], deg_sh.at[pl.ds(RP * NS, L)])

        @pl.loop(0, CH)
        def _(i):
            ones_v[i, :] = jnp.ones((L,), jnp.float32)

    plsc.subcore_barrier()

    pltpu.sync_copy(src3d.at[sub], idx1_v)
    pltpu.sync_copy(mid3d.at[sub], idx2_v)

    @pl.loop(0, NCK_S)
    def _(c):
        pltpu.async_copy(xsrc_sp.at[core].at[idx1_v.at[c]], rows_v, sem).wait()
        pltpu.sync_copy(rows_v, s_sh.at[idx2_v.at[c]], add=True)

        @pl.when(core == 0)
        def _():
            pltpu.sync_copy(ones_v, deg_sh.at[idx2_v.at[c]], add=True)

    plsc.subcore_barrier()

    _copy_out_rows(s_sh, s_out.at[core], sub)

    @pl.when(core == 0)
    def _():
        _copy_out_rows(deg_sh, deg_out, sub)


_RB = 1000


def _dense_body(xmid, xdst, s0, s1, deg16, W1, b12, Wq, bq, Wk, bk, Wv, bv,
                q_o, k_o, v_o):
    deg = deg16[:, 0:1]
    s = jnp.concatenate([s0[...], s1[...]], axis=1)
    agg = deg * (jnp.dot(xmid[...], W1[...], preferred_element_type=jnp.float32)
                 + b12[...]) + s
    q_o[...] = jnp.dot(xdst[...], Wq[...], preferred_element_type=jnp.float32) + bq[...]
    k_o[...] = jnp.dot(agg, Wk[...], preferred_element_type=jnp.float32) + bk[...]
    v_o[...] = jnp.dot(agg, Wv[...], preferred_element_type=jnp.float32) + bv[...]


def _dense_qkv(xmid, xdst, s0, s1, deg16, W1, b12, Wq, bq, Wk, bk, Wv, bv):
    row_spec = pl.BlockSpec((_RB, C), lambda i: (i, 0))
    half_spec = pl.BlockSpec((_RB, HC), lambda i: (i, 0))
    deg_spec = pl.BlockSpec((_RB, L), lambda i: (i, 0))
    w_spec = pl.BlockSpec((C, C), lambda i: (0, 0))
    b_spec = pl.BlockSpec((C,), lambda i: (0,))
    return pl.pallas_call(
        _dense_body,
        grid=(N_NODE // _RB,),
        in_specs=[row_spec, row_spec, half_spec, half_spec, deg_spec,
                  w_spec, b_spec, w_spec, b_spec, w_spec, b_spec,
                  w_spec, b_spec],
        out_specs=[row_spec, row_spec, row_spec],
        out_shape=[jax.ShapeDtypeStruct((N_NODE, C), jnp.float32)] * 3,
    )(xmid, xdst, s0, s1, deg16, W1, b12, Wq, bq, Wk, bk, Wv, bv)


def _xw_body(x, W, o):
    o[...] = jnp.dot(x[...], W[...], preferred_element_type=jnp.float32)


def _xw(x, W):
    row_spec = pl.BlockSpec((_RB, C), lambda i: (i, 0))
    w_spec = pl.BlockSpec((C, C), lambda i: (0, 0))
    return pl.pallas_call(
        _xw_body,
        grid=(N_NODE // _RB,),
        in_specs=[row_spec, w_spec],
        out_specs=row_spec,
        out_shape=jax.ShapeDtypeStruct((N_NODE, C), jnp.float32),
    )(x, W)


@functools.partial(
    pl.kernel,
    out_type=[
        jax.ShapeDtypeStruct((NW, NCK_W, CH), jnp.float32),
        jax.ShapeDtypeStruct((NW, 1, L), jnp.float32),
        jax.ShapeDtypeStruct((NW, 1, L), jnp.float32),
    ],
    mesh=_mesh,
    compiler_params=_sc_params_nl,
    scratch_types=[
        pltpu.VMEM((NCK_W, CH), jnp.int32),
        pltpu.VMEM((NCK_W, CH), jnp.int32),
        pltpu.VMEM((CH, C), jnp.float32),
        pltpu.VMEM((CH, C), jnp.float32),
        pltpu.VMEM((CH, C), jnp.float32),
        pltpu.VMEM((CH, C), jnp.float32),
        pltpu.VMEM((NCK_W, CH), jnp.float32),
        pltpu.VMEM((CH // L, L * L), jnp.float32),
        pltpu.VMEM((1, L), jnp.float32),
        pltpu.VMEM((1, L), jnp.float32),
        pltpu.SemaphoreType.DMA,
        pltpu.SemaphoreType.DMA,
        pltpu.SemaphoreType.DMA,
        pltpu.SemaphoreType.DMA,
    ],
)
def _logits_kernel(mid3d, dst3d, qtab, ktab, lg_out, mx_out, se_out,
                   idxm_v, idxd_v, q_b0, q_b1, k_b0, k_b1, lg_v, vacc,
                   mstat, sstat, qs0, qs1, ks0, ks1):
    core = lax.axis_index("c")
    sub = lax.axis_index("s")
    wid = sub * NC + core
    lane = jnp.arange(L, dtype=jnp.int32)
    q_bufs = (q_b0, q_b1)
    k_bufs = (k_b0, k_b1)
    qsems = (qs0, qs1)
    ksems = (ks0, ks1)

    pltpu.sync_copy(mid3d.at[wid], idxm_v)
    pltpu.sync_copy(dst3d.at[wid], idxd_v)

    def _start(c, b):
        pltpu.async_copy(qtab.at[idxd_v.at[c]], q_bufs[b], qsems[b])
        pltpu.async_copy(ktab.at[idxm_v.at[c]], k_bufs[b], ksems[b])

    def _wait(b):
        pltpu.make_async_copy(qtab.at[idxd_v.at[0]], q_bufs[b],
                              qsems[b]).wait()
        pltpu.make_async_copy(ktab.at[idxm_v.at[0]], k_bufs[b],
                              ksems[b]).wait()

    def _compute(c, b):
        q_rows = q_bufs[b]
        k_rows = k_bufs[b]
        for e in range(CH):
            acc = jnp.zeros((L,), jnp.float32)
            for j in range(C // L):
                acc = acc + (q_rows[e, pl.ds(j * L, L)]
                             * k_rows[e, pl.ds(j * L, L)])
            vacc[e // L, pl.ds((e % L) * L, L)] = acc
        for g in range(CH // L):
            rowsel = jnp.full((L,), g, jnp.int32)
            dots = jnp.zeros((L,), jnp.float32)
            for j in range(L):
                colsel = lane * L + j
                dots = dots + plsc.load_gather(vacc, [rowsel, colsel])
            lg_v[c, pl.ds(g * L, L)] = dots * INV_SQRT_C

    _start(0, 0)
    _start(1, 1)

    @pl.loop(0, NCK_W - 1, step=2)
    def _(c):
        for b in range(2):
            _wait(b)
            _compute(c + b, b)

            @pl.when(c + 2 + b < NCK_W)
            def _():
                _start(c + 2 + b, b)

    _wait((NCK_W - 1) % 2)
    _compute(NCK_W - 1, (NCK_W - 1) % 2)

    def mx_body(c, m):
        for g in range(CH // L):
            m = jnp.maximum(m, lg_v[c, pl.ds(g * L, L)])
        return m

    m_vec = lax.fori_loop(0, NCK_W, mx_body,
                          jnp.full((L,), -jnp.inf, jnp.float32))
    m_loc = jnp.max(m_vec)

    def se_body(c, s):
        for g in range(CH // L):
            s = s + jnp.exp(lg_v[c, pl.ds(g * L, L)] - m_loc)
        return s

    s_vec = lax.fori_loop(0, NCK_W, se_body, jnp.zeros((L,), jnp.float32))
    s_loc = jnp.sum(s_vec)

    mstat[0, :] = jnp.zeros((L,), jnp.float32) + m_loc
    sstat[0, :] = jnp.zeros((L,), jnp.float32) + s_loc
    pltpu.sync_copy(lg_v, lg_out.at[wid])
    pltpu.sync_copy(mstat, mx_out.at[wid])
    pltpu.sync_copy(sstat, se_out.at[wid])


@functools.partial(
    pl.kernel,
    out_type=jax.ShapeDtypeStruct((NC, N_NODE, HC), jnp.float32),
    mesh=_mesh,
    compiler_params=_sc_params,
    scratch_types=[
        pltpu.VMEM((NCK_S, CH), jnp.int32),
        pltpu.VMEM((NCK_S, CH), jnp.int32),
        pltpu.VMEM((NCK_S, CH), jnp.float32),
        pltpu.VMEM((CH, HC), jnp.float32),
        pltpu.VMEM((NW, 1, L), jnp.float32),
        pltpu.VMEM((NW, 1, L), jnp.float32),
        pltpu.VMEM((ZR, HC), jnp.float32),
        pltpu.VMEM_SHARED((N_NODE, HC), jnp.float32),
        pltpu.SemaphoreType.DMA,
    ],
)
def _attn_out_kernel(mid3d, dst3d, lg3d, vtab_sp, mx_in, se_in, out,
                     idxm_v, idxd_v, lg_v, v_rows, mx_v, se_v, zbuf,
                     o_sh, sem):
    core = lax.axis_index("c")
    sub = lax.axis_index("s")

    _zero_vmem_rows(zbuf, ZR, HC)
    _zero_shared_rows(zbuf, o_sh, sub)
    plsc.subcore_barrier()

    pltpu.sync_copy(mid3d.at[sub], idxm_v)
    pltpu.sync_copy(dst3d.at[sub], idxd_v)
    pltpu.sync_copy(lg3d.at[sub], lg_v)
    pltpu.sync_copy(mx_in, mx_v)
    pltpu.sync_copy(se_in, se_v)

    m_glob = mx_v[0, 0, :]
    for w in range(1, NW):
        m_glob = jnp.maximum(m_glob, mx_v[w, 0, :])
    z_vec = jnp.zeros((L,), jnp.float32)
    for w in range(NW):
        z_vec = z_vec + jnp.exp(mx_v[w, 0, :] - m_glob) * se_v[w, 0, :]
    inv_z = 1.0 / z_vec

    @pl.loop(0, NCK_S)
    def _(c):
        pltpu.async_copy(vtab_sp.at[core].at[idxm_v.at[c]], v_rows, sem).wait()
        for g in range(CH // L):
            a_g = jnp.exp(lg_v[c, pl.ds(g * L, L)] - m_glob) * inv_z
            for e in range(L):
                ee = g * L + e
                a_e = a_g[e]
                for j in range(HC // L):
                    v_rows[ee, pl.ds(j * L, L)] = v_rows[ee, pl.ds(j * L, L)] * a_e
        pltpu.sync_copy(v_rows, o_sh.at[idxd_v.at[c]], add=True)

    plsc.subcore_barrier()
    _copy_out_rows(o_sh, out.at[core], sub)


def _concat_body(a, b, o):
    o[...] = jnp.concatenate([a[...], b[...]], axis=1)


def _concat_halves(a, b):
    half_spec = pl.BlockSpec((_RB, HC), lambda i: (i, 0))
    return pl.pallas_call(
        _concat_body,
        grid=(N_NODE // _RB,),
        in_specs=[half_spec, half_spec],
        out_specs=pl.BlockSpec((_RB, C), lambda i: (i, 0)),
        out_shape=jax.ShapeDtypeStruct((N_NODE, C), jnp.float32),
    )(a, b)


def kernel(x_src, x_mid, x_dst, edge_index_1, edge_index_2,
           W1, b1, W2, b2, Wq, bq, Wk, bk, Wv, bv):
    src_s = edge_index_1[0].astype(jnp.int32).reshape(NS, NCK_S, CH)
    mid_s = edge_index_1[1].astype(jnp.int32).reshape(NS, NCK_S, CH)
    mid_w2 = edge_index_2[0].astype(jnp.int32).reshape(NW, NCK_W, CH)
    dst_w2 = edge_index_2[1].astype(jnp.int32).reshape(NW, NCK_W, CH)
    mid_s2 = edge_index_2[0].astype(jnp.int32).reshape(NS, NCK_S, CH)
    dst_s2 = edge_index_2[1].astype(jnp.int32).reshape(NS, NCK_S, CH)

    t2 = _xw(x_src, W2)
    t2_sp = jnp.stack([t2[:, :HC], t2[:, HC:]])

    s_halves, deg16 = _seg_sum_kernel(src_s, mid_s, t2_sp)

    q_tab, k_tab, v_tab = _dense_qkv(
        x_mid, x_dst, s_halves[0], s_halves[1], deg16,
        W1, b1 + b2, Wq, bq, Wk, bk, Wv, bv)

    lg3d, mx, se = _logits_kernel(mid_w2, dst_w2, q_tab, k_tab)

    vtab_sp = jnp.stack([v_tab[:, :HC], v_tab[:, HC:]])
    lg_s = lg3d.reshape(NS, NCK_S, CH)

    out_halves = _attn_out_kernel(mid_s2, dst_s2, lg_s, vtab_sp, mx, se)
    return _concat_halves(out_halves[0], out_halves[1])


# speedup vs baseline: 2.0866x; 1.2904x over previous
import functools
import jax
import jax.numpy as jnp
from jax import lax
from jax.experimental import pallas as pl
from jax.experimental.pallas import tpu as pltpu, tpu_sc as plsc

C = 128
HC = C // 2
N_NODE = 10000
N_EDGE = 320000
NC, NS, L = 2, 16, 16
NW = NC * NS
CH = 80
EPW = N_EDGE // NW
NCK_W = EPW // CH
EPS = N_EDGE // NS
NCK_S = EPS // CH
RP = 624
ZR = 208
INV_SQRT_C = 1.0 / (C ** 0.5)

_mesh = plsc.VectorSubcoreMesh(core_axis_name="c", subcore_axis_name="s")
_sc_params = pltpu.CompilerParams(use_tc_tiling_on_sc=False, has_side_effects=True)
_sc_params_nl = pltpu.CompilerParams(use_tc_tiling_on_sc=False,
                                     needs_layout_passes=False,
                                     has_side_effects=True)


def _zero_vmem_rows(ref, nrows, ncols):
    @pl.loop(0, nrows)
    def _(i):
        for j in range(ncols // L):
            ref[i, pl.ds(j * L, L)] = jnp.zeros((L,), jnp.float32)


def _zero_shared_rows(zbuf, sh_ref, sub):
    for t in range(RP // ZR):
        pltpu.sync_copy(zbuf, sh_ref.at[pl.ds(sub * RP + t * ZR, ZR)])

    @pl.when(sub == NS - 1)
    def _():
        pltpu.sync_copy(zbuf.at[pl.ds(0, L)], sh_ref.at[pl.ds(RP * NS, L)])


def _copy_out_rows(sh_ref, out_ref, sub):
    pltpu.sync_copy(sh_ref.at[pl.ds(sub * RP, RP)],
                    out_ref.at[pl.ds(sub * RP, RP)])

    @pl.when(sub == NS - 1)
    def _():
        pltpu.sync_copy(sh_ref.at[pl.ds(RP * NS, L)],
                        out_ref.at[pl.ds(RP * NS, L)])


@functools.partial(
    pl.kernel,
    out_type=[
        jax.ShapeDtypeStruct((NC, N_NODE, HC), jnp.float32),
        jax.ShapeDtypeStruct((N_NODE, L), jnp.float32),
    ],
    mesh=_mesh,
    compiler_params=_sc_params,
    scratch_types=[
        pltpu.VMEM((NCK_S, CH), jnp.int32),
        pltpu.VMEM((NCK_S, CH), jnp.int32),
        pltpu.VMEM((CH, HC), jnp.float32),
        pltpu.VMEM((CH, HC), jnp.float32),
        pltpu.VMEM((CH, L), jnp.float32),
        pltpu.VMEM((ZR, HC), jnp.float32),
        pltpu.VMEM((ZR, L), jnp.float32),
        pltpu.VMEM_SHARED((N_NODE, HC), jnp.float32),
        pltpu.VMEM_SHARED((N_NODE, L), jnp.float32),
        pltpu.SemaphoreType.DMA,
        pltpu.SemaphoreType.DMA,
    ],
)
def _seg_sum_kernel(src3d, mid3d, xsrc_sp, s_out, deg_out,
                    idx1_v, idx2_v, rows_v0, rows_v1, ones_v, zbuf, degz,
                    s_sh, deg_sh, sem0, sem1):
    core = lax.axis_index("c")
    sub = lax.axis_index("s")

    _zero_vmem_rows(zbuf, ZR, HC)
    _zero_shared_rows(zbuf, s_sh, sub)

    @pl.when(core == 0)
    def _():
        _zero_vmem_rows(degz, ZR, L)
        for t in range(RP // ZR):
            pltpu.sync_copy(degz, deg_sh.at[pl.ds(sub * RP + t * ZR, ZR)])

        @pl.when(sub == NS - 1)
        def _():
            pltpu.sync_copy(degz.at[pl.ds(0, L)], deg_sh.at[pl.ds(RP * NS, L)])

        @pl.loop(0, CH)
        def _(i):
            ones_v[i, :] = jnp.ones((L,), jnp.float32)

    plsc.subcore_barrier()

    pltpu.sync_copy(src3d.at[sub], idx1_v)
    pltpu.sync_copy(mid3d.at[sub], idx2_v)

    rows_bufs = (rows_v0, rows_v1)
    sems = (sem0, sem1)

    def _start(c, b):
        pltpu.async_copy(xsrc_sp.at[core].at[idx1_v.at[c]], rows_bufs[b],
                         sems[b])

    _start(0, 0)
    _start(1, 1)

    @pl.loop(0, NCK_S, step=2)
    def _(c):
        for b in range(2):
            pltpu.make_async_copy(xsrc_sp.at[core].at[idx1_v.at[0]],
                                  rows_bufs[b], sems[b]).wait()
            pltpu.sync_copy(rows_bufs[b], s_sh.at[idx2_v.at[c + b]], add=True)

            @pl.when(c + 2 + b < NCK_S)
            def _():
                _start(c + 2 + b, b)

            @pl.when(core == 0)
            def _():
                pltpu.sync_copy(ones_v, deg_sh.at[idx2_v.at[c + b]], add=True)

    plsc.subcore_barrier()

    _copy_out_rows(s_sh, s_out.at[core], sub)

    @pl.when(core == 0)
    def _():
        _copy_out_rows(deg_sh, deg_out, sub)


_RB = 1000


def _dense_body(xmid, xdst, s0, s1, deg16, W1, b12, Wq, bq, Wk, bk, Wv, bv,
                q_o, k_o, v_o):
    deg = deg16[:, 0:1]
    s = jnp.concatenate([s0[...], s1[...]], axis=1)
    agg = deg * (jnp.dot(xmid[...], W1[...], preferred_element_type=jnp.float32)
                 + b12[...]) + s
    q_o[...] = jnp.dot(xdst[...], Wq[...], preferred_element_type=jnp.float32) + bq[...]
    k_o[...] = jnp.dot(agg, Wk[...], preferred_element_type=jnp.float32) + bk[...]
    v_o[...] = jnp.dot(agg, Wv[...], preferred_element_type=jnp.float32) + bv[...]


def _dense_qkv(xmid, xdst, s0, s1, deg16, W1, b12, Wq, bq, Wk, bk, Wv, bv):
    row_spec = pl.BlockSpec((_RB, C), lambda i: (i, 0))
    half_spec = pl.BlockSpec((_RB, HC), lambda i: (i, 0))
    deg_spec = pl.BlockSpec((_RB, L), lambda i: (i, 0))
    w_spec = pl.BlockSpec((C, C), lambda i: (0, 0))
    b_spec = pl.BlockSpec((C,), lambda i: (0,))
    return pl.pallas_call(
        _dense_body,
        grid=(N_NODE // _RB,),
        in_specs=[row_spec, row_spec, half_spec, half_spec, deg_spec,
                  w_spec, b_spec, w_spec, b_spec, w_spec, b_spec,
                  w_spec, b_spec],
        out_specs=[row_spec, row_spec, row_spec],
        out_shape=[jax.ShapeDtypeStruct((N_NODE, C), jnp.float32)] * 3,
    )(xmid, xdst, s0, s1, deg16, W1, b12, Wq, bq, Wk, bk, Wv, bv)


def _xw_body(x, W, o):
    o[...] = jnp.dot(x[...], W[...], preferred_element_type=jnp.float32)


def _xw(x, W):
    row_spec = pl.BlockSpec((_RB, C), lambda i: (i, 0))
    w_spec = pl.BlockSpec((C, C), lambda i: (0, 0))
    return pl.pallas_call(
        _xw_body,
        grid=(N_NODE // _RB,),
        in_specs=[row_spec, w_spec],
        out_specs=row_spec,
        out_shape=jax.ShapeDtypeStruct((N_NODE, C), jnp.float32),
    )(x, W)


@functools.partial(
    pl.kernel,
    out_type=[
        jax.ShapeDtypeStruct((NW, NCK_W, CH), jnp.float32),
        jax.ShapeDtypeStruct((NW, 1, L), jnp.float32),
        jax.ShapeDtypeStruct((NW, 1, L), jnp.float32),
    ],
    mesh=_mesh,
    compiler_params=_sc_params_nl,
    scratch_types=[
        pltpu.VMEM((NCK_W, CH), jnp.int32),
        pltpu.VMEM((NCK_W, CH), jnp.int32),
        pltpu.VMEM((CH, C), jnp.float32),
        pltpu.VMEM((CH, C), jnp.float32),
        pltpu.VMEM((CH, C), jnp.float32),
        pltpu.VMEM((CH, C), jnp.float32),
        pltpu.VMEM((NCK_W, CH), jnp.float32),
        pltpu.VMEM((CH // L, L * L), jnp.float32),
        pltpu.VMEM((1, L), jnp.float32),
        pltpu.VMEM((1, L), jnp.float32),
        pltpu.SemaphoreType.DMA,
        pltpu.SemaphoreType.DMA,
        pltpu.SemaphoreType.DMA,
        pltpu.SemaphoreType.DMA,
    ],
)
def _logits_kernel(mid3d, dst3d, qtab, ktab, lg_out, mx_out, se_out,
                   idxm_v, idxd_v, q_b0, q_b1, k_b0, k_b1, lg_v, vacc,
                   mstat, sstat, qs0, qs1, ks0, ks1):
    core = lax.axis_index("c")
    sub = lax.axis_index("s")
    wid = sub * NC + core
    lane = jnp.arange(L, dtype=jnp.int32)
    q_bufs = (q_b0, q_b1)
    k_bufs = (k_b0, k_b1)
    qsems = (qs0, qs1)
    ksems = (ks0, ks1)

    pltpu.sync_copy(mid3d.at[wid], idxm_v)
    pltpu.sync_copy(dst3d.at[wid], idxd_v)

    def _start(c, b):
        pltpu.async_copy(qtab.at[idxd_v.at[c]], q_bufs[b], qsems[b])
        pltpu.async_copy(ktab.at[idxm_v.at[c]], k_bufs[b], ksems[b])

    def _wait(b):
        pltpu.make_async_copy(qtab.at[idxd_v.at[0]], q_bufs[b],
                              qsems[b]).wait()
        pltpu.make_async_copy(ktab.at[idxm_v.at[0]], k_bufs[b],
                              ksems[b]).wait()

    def _compute(c, b):
        q_rows = q_bufs[b]
        k_rows = k_bufs[b]
        for e in range(CH):
            acc = jnp.zeros((L,), jnp.float32)
            for j in range(C // L):
                acc = acc + (q_rows[e, pl.ds(j * L, L)]
                             * k_rows[e, pl.ds(j * L, L)])
            vacc[e // L, pl.ds((e % L) * L, L)] = acc
        for g in range(CH // L):
            rowsel = jnp.full((L,), g, jnp.int32)
            dots = jnp.zeros((L,), jnp.float32)
            for j in range(L):
                colsel = lane * L + j
                dots = dots + plsc.load_gather(vacc, [rowsel, colsel])
            lg_v[c, pl.ds(g * L, L)] = dots * INV_SQRT_C

    _start(0, 0)
    _start(1, 1)

    @pl.loop(0, NCK_W - 1, step=2)
    def _(c):
        for b in range(2):
            _wait(b)
            _compute(c + b, b)

            @pl.when(c + 2 + b < NCK_W)
            def _():
                _start(c + 2 + b, b)

    _wait((NCK_W - 1) % 2)
    _compute(NCK_W - 1, (NCK_W - 1) % 2)

    def mx_body(c, m):
        for g in range(CH // L):
            m = jnp.maximum(m, lg_v[c, pl.ds(g * L, L)])
        return m

    m_vec = lax.fori_loop(0, NCK_W, mx_body,
                          jnp.full((L,), -jnp.inf, jnp.float32))
    m_loc = jnp.max(m_vec)

    def se_body(c, s):
        for g in range(CH // L):
            s = s + jnp.exp(lg_v[c, pl.ds(g * L, L)] - m_loc)
        return s

    s_vec = lax.fori_loop(0, NCK_W, se_body, jnp.zeros((L,), jnp.float32))
    s_loc = jnp.sum(s_vec)

    mstat[0, :] = jnp.zeros((L,), jnp.float32) + m_loc
    sstat[0, :] = jnp.zeros((L,), jnp.float32) + s_loc
    pltpu.sync_copy(lg_v, lg_out.at[wid])
    pltpu.sync_copy(mstat, mx_out.at[wid])
    pltpu.sync_copy(sstat, se_out.at[wid])


@functools.partial(
    pl.kernel,
    out_type=jax.ShapeDtypeStruct((NC, N_NODE, HC), jnp.float32),
    mesh=_mesh,
    compiler_params=_sc_params,
    scratch_types=[
        pltpu.VMEM((NCK_S, CH), jnp.int32),
        pltpu.VMEM((NCK_S, CH), jnp.int32),
        pltpu.VMEM((NCK_S, CH), jnp.float32),
        pltpu.VMEM((CH, HC), jnp.float32),
        pltpu.VMEM((CH, HC), jnp.float32),
        pltpu.VMEM((NW, 1, L), jnp.float32),
        pltpu.VMEM((NW, 1, L), jnp.float32),
        pltpu.VMEM((ZR, HC), jnp.float32),
        pltpu.VMEM_SHARED((N_NODE, HC), jnp.float32),
        pltpu.SemaphoreType.DMA,
        pltpu.SemaphoreType.DMA,
    ],
)
def _attn_out_kernel(mid3d, dst3d, lg3d, vtab_sp, mx_in, se_in, out,
                     idxm_v, idxd_v, lg_v, v_rows0, v_rows1, mx_v, se_v, zbuf,
                     o_sh, sem0, sem1):
    core = lax.axis_index("c")
    sub = lax.axis_index("s")

    _zero_vmem_rows(zbuf, ZR, HC)
    _zero_shared_rows(zbuf, o_sh, sub)
    plsc.subcore_barrier()

    pltpu.sync_copy(mid3d.at[sub], idxm_v)
    pltpu.sync_copy(dst3d.at[sub], idxd_v)
    pltpu.sync_copy(lg3d.at[sub], lg_v)
    pltpu.sync_copy(mx_in, mx_v)
    pltpu.sync_copy(se_in, se_v)

    m_glob = mx_v[0, 0, :]
    for w in range(1, NW):
        m_glob = jnp.maximum(m_glob, mx_v[w, 0, :])
    z_vec = jnp.zeros((L,), jnp.float32)
    for w in range(NW):
        z_vec = z_vec + jnp.exp(mx_v[w, 0, :] - m_glob) * se_v[w, 0, :]
    inv_z = 1.0 / z_vec

    v_bufs = (v_rows0, v_rows1)
    sems = (sem0, sem1)

    def _start(c, b):
        pltpu.async_copy(vtab_sp.at[core].at[idxm_v.at[c]], v_bufs[b],
                         sems[b])

    _start(0, 0)
    _start(1, 1)

    @pl.loop(0, NCK_S, step=2)
    def _(c):
        for b in range(2):
            v_rows = v_bufs[b]
            pltpu.make_async_copy(vtab_sp.at[core].at[idxm_v.at[0]],
                                  v_rows, sems[b]).wait()
            for g in range(CH // L):
                a_g = jnp.exp(lg_v[c + b, pl.ds(g * L, L)] - m_glob) * inv_z
                for e in range(L):
                    ee = g * L + e
                    a_e = a_g[e]
                    for j in range(HC // L):
                        v_rows[ee, pl.ds(j * L, L)] = (
                            v_rows[ee, pl.ds(j * L, L)] * a_e)
            pltpu.sync_copy(v_rows, o_sh.at[idxd_v.at[c + b]], add=True)

            @pl.when(c + 2 + b < NCK_S)
            def _():
                _start(c + 2 + b, b)

    plsc.subcore_barrier()
    _copy_out_rows(o_sh, out.at[core], sub)


def _concat_body(a, b, o):
    o[...] = jnp.concatenate([a[...], b[...]], axis=1)


def _concat_halves(a, b):
    half_spec = pl.BlockSpec((_RB, HC), lambda i: (i, 0))
    return pl.pallas_call(
        _concat_body,
        grid=(N_NODE // _RB,),
        in_specs=[half_spec, half_spec],
        out_specs=pl.BlockSpec((_RB, C), lambda i: (i, 0)),
        out_shape=jax.ShapeDtypeStruct((N_NODE, C), jnp.float32),
    )(a, b)


def kernel(x_src, x_mid, x_dst, edge_index_1, edge_index_2,
           W1, b1, W2, b2, Wq, bq, Wk, bk, Wv, bv):
    src_s = edge_index_1[0].astype(jnp.int32).reshape(NS, NCK_S, CH)
    mid_s = edge_index_1[1].astype(jnp.int32).reshape(NS, NCK_S, CH)
    mid_w2 = edge_index_2[0].astype(jnp.int32).reshape(NW, NCK_W, CH)
    dst_w2 = edge_index_2[1].astype(jnp.int32).reshape(NW, NCK_W, CH)
    mid_s2 = edge_index_2[0].astype(jnp.int32).reshape(NS, NCK_S, CH)
    dst_s2 = edge_index_2[1].astype(jnp.int32).reshape(NS, NCK_S, CH)

    t2 = _xw(x_src, W2)
    t2_sp = jnp.stack([t2[:, :HC], t2[:, HC:]])

    s_halves, deg16 = _seg_sum_kernel(src_s, mid_s, t2_sp)

    q_tab, k_tab, v_tab = _dense_qkv(
        x_mid, x_dst, s_halves[0], s_halves[1], deg16,
        W1, b1 + b2, Wq, bq, Wk, bk, Wv, bv)

    lg3d, mx, se = _logits_kernel(mid_w2, dst_w2, q_tab, k_tab)

    vtab_sp = jnp.stack([v_tab[:, :HC], v_tab[:, HC:]])
    lg_s = lg3d.reshape(NS, NCK_S, CH)

    out_halves = _attn_out_kernel(mid_s2, dst_s2, lg_s, vtab_sp, mx, se)
    return _concat_halves(out_halves[0], out_halves[1])
